# Initial kernel scaffold; baseline (speedup 1.0000x reference)
import functools

import jax
import jax.numpy as jnp
from jax import lax
from jax.experimental import pallas as pl
from jax.experimental.pallas import tpu as pltpu
from jax.experimental.pallas import tpu_sc as plsc


def _sc_body(pos_hbm, neg_hbm, out_hbm, pv, nv, ov):
    cid = lax.axis_index("c")
    sid = lax.axis_index("s")

    @pl.when(jnp.logical_and(cid == 0, sid == 0))
    def _():
        pltpu.sync_copy(pos_hbm.at[pl.ds(0, 64)], pv)
        pltpu.sync_copy(neg_hbm.at[pl.ds(0, 64)], nv)

        i0 = lax.iota(jnp.int32, 16) * 4
        i1 = i0 + 1
        i2 = i0 + 2
        i3 = i0 + 3
        a = (plsc.load_gather(pv, [i0]) + plsc.load_gather(pv, [i1])
             + plsc.load_gather(nv, [i0]) + plsc.load_gather(nv, [i1]))
        b = (plsc.load_gather(pv, [i2]) + plsc.load_gather(pv, [i3])
             + plsc.load_gather(nv, [i2]) + plsc.load_gather(nv, [i3]))

        m = jnp.maximum(a, b)
        u = jnp.exp(-jnp.abs(a - b)) + jnp.float32(1e-15)
        z = u / (u + jnp.float32(2.0))
        z2 = z * z
        p = jnp.float32(1.0 / 11.0)
        p = jnp.float32(1.0 / 9.0) + z2 * p
        p = jnp.float32(1.0 / 7.0) + z2 * p
        p = jnp.float32(1.0 / 5.0) + z2 * p
        p = jnp.float32(1.0 / 3.0) + z2 * p
        p = jnp.float32(1.0) + z2 * p
        log1p_u = jnp.float32(2.0) * z * p

        ov[...] = m + log1p_u
        pltpu.sync_copy(ov, out_hbm)


def kernel(pos, neg):
    mesh = plsc.VectorSubcoreMesh(core_axis_name="c", subcore_axis_name="s")
    run = functools.partial(
        pl.kernel,
        mesh=mesh,
        out_type=jax.ShapeDtypeStruct((16,), jnp.float32),
        scratch_types=[
            pltpu.VMEM((64,), jnp.float32),
            pltpu.VMEM((64,), jnp.float32),
            pltpu.VMEM((16,), jnp.float32),
        ],
    )(_sc_body)
    return run(pos, neg)


# speedup vs baseline: 2.9209x; 2.9209x over previous
import functools

import jax
import jax.numpy as jnp
from jax import lax
from jax.experimental import pallas as pl
from jax.experimental.pallas import tpu as pltpu
from jax.experimental.pallas import tpu_sc as plsc


def _sc_body(pos_hbm, neg_hbm, out_hbm, pv, nv, ov):
    cid = lax.axis_index("c")
    sid = lax.axis_index("s")

    @pl.when(jnp.logical_and(cid == 0, sid == 0))
    def _():
        pltpu.sync_copy(pos_hbm.at[pl.ds(0, 64)], pv)
        pltpu.sync_copy(neg_hbm.at[pl.ds(0, 64)], nv)

        i0 = lax.iota(jnp.int32, 16) * 4
        i1 = i0 + 1
        i2 = i0 + 2
        i3 = i0 + 3
        a = (plsc.load_gather(pv, [i0]) + plsc.load_gather(pv, [i1])
             + plsc.load_gather(nv, [i0]) + plsc.load_gather(nv, [i1]))
        b = (plsc.load_gather(pv, [i2]) + plsc.load_gather(pv, [i3])
             + plsc.load_gather(nv, [i2]) + plsc.load_gather(nv, [i3]))

        m = jnp.maximum(a, b)
        u = jnp.exp(-jnp.abs(a - b)) + jnp.float32(1e-15)
        z = u / (u + jnp.float32(2.0))
        z2 = z * z
        p = jnp.float32(1.0 / 11.0)
        p = jnp.float32(1.0 / 9.0) + z2 * p
        p = jnp.float32(1.0 / 7.0) + z2 * p
        p = jnp.float32(1.0 / 5.0) + z2 * p
        p = jnp.float32(1.0 / 3.0) + z2 * p
        p = jnp.float32(1.0) + z2 * p
        log1p_u = jnp.float32(2.0) * z * p

        ov[...] = m + log1p_u
        pltpu.sync_copy(ov, out_hbm)


def kernel(pos, neg):
    mesh = plsc.VectorSubcoreMesh(core_axis_name="c", subcore_axis_name="s")
    run = functools.partial(
        pl.kernel,
        mesh=mesh,
        out_type=jax.ShapeDtypeStruct((16,), jnp.float32),
        compiler_params=pltpu.CompilerParams(needs_layout_passes=False),
        scratch_types=[
            pltpu.VMEM((64,), jnp.float32),
            pltpu.VMEM((64,), jnp.float32),
            pltpu.VMEM((16,), jnp.float32),
        ],
    )(_sc_body)
    return run(pos, neg)


# speedup vs baseline: 3.2346x; 1.1074x over previous
import functools

import jax
import jax.numpy as jnp
from jax import lax
from jax.experimental import pallas as pl
from jax.experimental.pallas import tpu as pltpu
from jax.experimental.pallas import tpu_sc as plsc


def _sc_body(pos_hbm, neg_hbm, out_hbm, pv, nv, ov, sem_p, sem_n):
    cp = pltpu.async_copy(pos_hbm.at[pl.ds(0, 64)], pv, sem_p)
    cn = pltpu.async_copy(neg_hbm.at[pl.ds(0, 64)], nv, sem_n)
    cp.wait()
    cn.wait()

    i0 = lax.iota(jnp.int32, 16) * 4
    i1 = i0 + 1
    i2 = i0 + 2
    i3 = i0 + 3
    a = (plsc.load_gather(pv, [i0]) + plsc.load_gather(pv, [i1])
         + plsc.load_gather(nv, [i0]) + plsc.load_gather(nv, [i1]))
    b = (plsc.load_gather(pv, [i2]) + plsc.load_gather(pv, [i3])
         + plsc.load_gather(nv, [i2]) + plsc.load_gather(nv, [i3]))

    m = jnp.maximum(a, b)
    u = jnp.exp(-jnp.abs(a - b)) + jnp.float32(1e-15)
    z = u / (u + jnp.float32(2.0))
    z2 = z * z
    p = jnp.float32(1.0 / 11.0)
    p = jnp.float32(1.0 / 9.0) + z2 * p
    p = jnp.float32(1.0 / 7.0) + z2 * p
    p = jnp.float32(1.0 / 5.0) + z2 * p
    p = jnp.float32(1.0 / 3.0) + z2 * p
    p = jnp.float32(1.0) + z2 * p
    log1p_u = jnp.float32(2.0) * z * p

    ov[...] = m + log1p_u
    pltpu.sync_copy(ov, out_hbm)


def kernel(pos, neg):
    mesh = plsc.VectorSubcoreMesh(
        core_axis_name="c", subcore_axis_name="s", num_cores=1, num_subcores=1
    )
    run = functools.partial(
        pl.kernel,
        mesh=mesh,
        out_type=jax.ShapeDtypeStruct((16,), jnp.float32),
        compiler_params=pltpu.CompilerParams(needs_layout_passes=False),
        scratch_types=[
            pltpu.VMEM((64,), jnp.float32),
            pltpu.VMEM((64,), jnp.float32),
            pltpu.VMEM((16,), jnp.float32),
            pltpu.SemaphoreType.DMA,
            pltpu.SemaphoreType.DMA,
        ],
    )(_sc_body)
    return run(pos, neg)
